# baseline (speedup 1.0000x reference)
import functools

import jax
import jax.numpy as jnp
from jax import lax
from jax.experimental import pallas as pl
from jax.experimental.pallas import tpu as pltpu
from jax.experimental.pallas import tpu_sc as plsc

_V = 100000
_E = 64
_B = 4096
_H = 200

_NC = 2
_NS = 16
_NW = _NC * _NS
_LANES = 16

_ROWS_PER_W = _B // _NW
_GROUPS = _ROWS_PER_W // _LANES


def _stage1_body(b_ref, tblT_lo_ref, tblT_hi_ref, w_lo_ref, w_hi_ref, out_ref):
    out_ref[...] = (
        jnp.sum(tblT_lo_ref[...] * w_lo_ref[...], axis=0)
        + jnp.sum(tblT_hi_ref[...] * w_hi_ref[...], axis=0)
        + b_ref[0]
    )


def _stage1(tableT, w_scaled, b_scaled):
    blk = 25600
    he = _E // 2
    return pl.pallas_call(
        _stage1_body,
        grid=(pl.cdiv(_V, blk),),
        in_specs=[
            pl.BlockSpec(memory_space=pltpu.SMEM),
            pl.BlockSpec((he, blk), lambda i: (0, i)),
            pl.BlockSpec((he, blk), lambda i: (1, i)),
            pl.BlockSpec((he, 1), lambda i: (0, 0)),
            pl.BlockSpec((he, 1), lambda i: (1, 0)),
        ],
        out_specs=pl.BlockSpec((blk,), lambda i: (i,)),
        out_shape=jax.ShapeDtypeStruct((_V,), jnp.float32),
    )(b_scaled, tableT, tableT, w_scaled, w_scaled)


def _sc_body(t_hbm, x4_hbm, out_hbm, t_v, idx_v, res_v, sem_t, sem_x):
    wid = lax.axis_index("s") * _NC + lax.axis_index("c")
    base_row = wid * _ROWS_PER_W

    ct = pltpu.async_copy(t_hbm, t_v, sem_t)
    cx = pltpu.async_copy(x4_hbm.at[:, wid], idx_v, sem_x)
    cx.wait()
    ct.wait()

    for g in range(_GROUPS):

        def tok_body(lh, acc, g=g):
            for ll in range(8):
                iv = idx_v[lh, ll, pl.ds(g * _LANES, _LANES)]
                acc = acc + plsc.load_gather(t_v, [iv])
            return acc

        z = lax.fori_loop(
            0, _H // 8, tok_body, jnp.zeros((_LANES,), jnp.float32)
        )
        res_v[pl.ds(g * _LANES, _LANES)] = 1.0 / (1.0 + jnp.exp(-z))

    pltpu.sync_copy(res_v, out_hbm.at[pl.ds(base_row, _ROWS_PER_W)])


@functools.cache
def _sc_stage2():
    return pl.kernel(
        _sc_body,
        out_type=jax.ShapeDtypeStruct((_B,), jnp.float32),
        mesh=plsc.VectorSubcoreMesh(
            core_axis_name="c", subcore_axis_name="s", num_cores=_NC, num_subcores=_NS
        ),
        scratch_types=[
            pltpu.VMEM((_V,), jnp.float32),
            pltpu.VMEM((_H // 8, 8, _ROWS_PER_W), jnp.int32),
            pltpu.VMEM((_ROWS_PER_W,), jnp.float32),
            pltpu.SemaphoreType.DMA,
            pltpu.SemaphoreType.DMA,
        ],
        compiler_params=pltpu.CompilerParams(
            use_tc_tiling_on_sc=False, needs_layout_passes=False
        ),
    )


@jax.jit
def kernel(x, table, W, b):
    w_scaled = (W.astype(jnp.float32) * (1.0 / _H)).reshape(_E, 1)
    b_scaled = b.astype(jnp.float32) * (1.0 / _H)
    t = _stage1(table.T, w_scaled, b_scaled)
    x4 = (
        x.astype(jnp.int32)
        .T.reshape(_H // 8, 8, _NW, _ROWS_PER_W)
        .transpose(0, 2, 1, 3)
    )
    y = _sc_stage2()(t, x4)
    return y.reshape(_B, 1)


# speedup vs baseline: 1.0456x; 1.0456x over previous
import functools

import jax
import jax.numpy as jnp
from jax import lax
from jax.experimental import pallas as pl
from jax.experimental.pallas import tpu as pltpu
from jax.experimental.pallas import tpu_sc as plsc

_V = 100000
_E = 64
_B = 4096
_H = 200

_NC = 2
_NS = 16
_NW = _NC * _NS
_LANES = 16

_ROWS_PER_W = _B // _NW
_GROUPS = _ROWS_PER_W // _LANES


def _stage1_body(b_ref, tblT_ref, w_ref, out_ref):
    out_ref[...] = jnp.sum(tblT_ref[...] * w_ref[...], axis=0) + b_ref[0]


def _stage1(tableT, w_scaled, b_scaled):
    blk = 25600
    return pl.pallas_call(
        _stage1_body,
        grid=(pl.cdiv(_V, blk),),
        in_specs=[
            pl.BlockSpec(memory_space=pltpu.SMEM),
            pl.BlockSpec((_E, blk), lambda i: (0, i)),
            pl.BlockSpec((_E, 1), lambda i: (0, 0)),
        ],
        out_specs=pl.BlockSpec((blk,), lambda i: (i,)),
        out_shape=jax.ShapeDtypeStruct((_V,), jnp.float32),
    )(b_scaled, tableT, w_scaled)


def _sc_body(t_hbm, x4_hbm, out_hbm, t_v, idx_v, res_v, sem_t, sem_x):
    wid = lax.axis_index("s") * _NC + lax.axis_index("c")

    @pl.when(wid < _NW // 2)
    def _():
        ct = pltpu.async_copy(t_hbm, t_v, sem_t)

        for half in range(2):
            blkid = 2 * wid + half
            cx = pltpu.async_copy(x4_hbm.at[:, blkid], idx_v, sem_x)
            cx.wait()
            if half == 0:
                ct.wait()

            for g in range(_GROUPS):

                def tok_body(lh, acc, g=g):
                    for ll in range(8):
                        iv = idx_v[lh, ll, pl.ds(g * _LANES, _LANES)]
                        acc = acc + plsc.load_gather(t_v, [iv])
                    return acc

                z = lax.fori_loop(
                    0, _H // 8, tok_body, jnp.zeros((_LANES,), jnp.float32)
                )
                res_v[pl.ds(g * _LANES, _LANES)] = 1.0 / (1.0 + jnp.exp(-z))

            pltpu.sync_copy(
                res_v, out_hbm.at[pl.ds(blkid * _ROWS_PER_W, _ROWS_PER_W)]
            )


@functools.cache
def _sc_stage2():
    return pl.kernel(
        _sc_body,
        out_type=jax.ShapeDtypeStruct((_B,), jnp.float32),
        mesh=plsc.VectorSubcoreMesh(
            core_axis_name="c", subcore_axis_name="s", num_cores=_NC, num_subcores=_NS
        ),
        scratch_types=[
            pltpu.VMEM((_V,), jnp.float32),
            pltpu.VMEM((_H // 8, 8, _ROWS_PER_W), jnp.int32),
            pltpu.VMEM((_ROWS_PER_W,), jnp.float32),
            pltpu.SemaphoreType.DMA,
            pltpu.SemaphoreType.DMA,
        ],
        compiler_params=pltpu.CompilerParams(
            use_tc_tiling_on_sc=False, needs_layout_passes=False
        ),
    )


@jax.jit
def kernel(x, table, W, b):
    w_scaled = (W.astype(jnp.float32) * (1.0 / _H)).reshape(_E, 1)
    b_scaled = b.astype(jnp.float32) * (1.0 / _H)
    t = _stage1(table.T, w_scaled, b_scaled)
    x4 = (
        x.astype(jnp.int32)
        .T.reshape(_H // 8, 8, _NW, _ROWS_PER_W)
        .transpose(0, 2, 1, 3)
    )
    y = _sc_stage2()(t, x4)
    return y.reshape(_B, 1)


# speedup vs baseline: 1.0787x; 1.0317x over previous
import functools

import jax
import jax.numpy as jnp
from jax import lax
from jax.experimental import pallas as pl
from jax.experimental.pallas import tpu as pltpu
from jax.experimental.pallas import tpu_sc as plsc

_V = 100000
_E = 64
_B = 4096
_H = 200

_NC = 2
_NS = 16
_NW = _NC * _NS
_LANES = 16

_ROWS_PER_W = _B // _NW
_GROUPS = _ROWS_PER_W // _LANES


def _stage1_body(b_ref, tblT_ref, w_ref, out_ref):
    out_ref[...] = jnp.sum(tblT_ref[...] * w_ref[...], axis=0) + b_ref[0]


def _stage1(tableT, w_scaled, b_scaled):
    blk = 25600
    return pl.pallas_call(
        _stage1_body,
        grid=(pl.cdiv(_V, blk),),
        in_specs=[
            pl.BlockSpec(memory_space=pltpu.SMEM),
            pl.BlockSpec((_E, blk), lambda i: (0, i)),
            pl.BlockSpec((_E, 1), lambda i: (0, 0)),
        ],
        out_specs=pl.BlockSpec((blk,), lambda i: (i,)),
        out_shape=jax.ShapeDtypeStruct((_V,), jnp.float32),
    )(b_scaled, tableT, w_scaled)


_CHUNK_ROWS = 64
_CHUNKS = 4


def _sc_body(t_hbm, x4_hbm, out_hbm, t_v, idx_a, idx_b, res_v, sem_t, sem_x):
    wid = lax.axis_index("s") * _NC + lax.axis_index("c")

    def chunk_src(c):
        return x4_hbm.at[:, 2 * wid + c // 2, :, pl.ds((c % 2) * _CHUNK_ROWS, _CHUNK_ROWS)]

    @pl.when(wid < _NW // 2)
    def _():
        bufs = [idx_a, idx_b]
        ct = pltpu.async_copy(t_hbm, t_v, sem_t)
        cps = [pltpu.async_copy(chunk_src(0), bufs[0], sem_x), None]

        for c in range(_CHUNKS):
            cps[c % 2].wait()
            if c == 0:
                ct.wait()
            if c + 1 < _CHUNKS:
                cps[(c + 1) % 2] = pltpu.async_copy(
                    chunk_src(c + 1), bufs[(c + 1) % 2], sem_x
                )
            buf = bufs[c % 2]

            for g in range(_CHUNK_ROWS // _LANES):

                def tok_body(lh, acc, g=g, buf=buf):
                    for ll in range(8):
                        iv = buf[lh, ll, pl.ds(g * _LANES, _LANES)]
                        acc = acc + plsc.load_gather(t_v, [iv])
                    return acc

                z = lax.fori_loop(
                    0, _H // 8, tok_body, jnp.zeros((_LANES,), jnp.float32)
                )
                res_v[pl.ds(g * _LANES, _LANES)] = 1.0 / (1.0 + jnp.exp(-z))

            pltpu.sync_copy(
                res_v, out_hbm.at[pl.ds((2 * wid) * _ROWS_PER_W + c * _CHUNK_ROWS, _CHUNK_ROWS)]
            )


@functools.cache
def _sc_stage2():
    return pl.kernel(
        _sc_body,
        out_type=jax.ShapeDtypeStruct((_B,), jnp.float32),
        mesh=plsc.VectorSubcoreMesh(
            core_axis_name="c", subcore_axis_name="s", num_cores=_NC, num_subcores=_NS
        ),
        scratch_types=[
            pltpu.VMEM((_V,), jnp.float32),
            pltpu.VMEM((_H // 8, 8, _CHUNK_ROWS), jnp.int32),
            pltpu.VMEM((_H // 8, 8, _CHUNK_ROWS), jnp.int32),
            pltpu.VMEM((_CHUNK_ROWS,), jnp.float32),
            pltpu.SemaphoreType.DMA,
            pltpu.SemaphoreType.DMA,
        ],
        compiler_params=pltpu.CompilerParams(
            use_tc_tiling_on_sc=False, needs_layout_passes=False
        ),
    )


@jax.jit
def kernel(x, table, W, b):
    w_scaled = (W.astype(jnp.float32) * (1.0 / _H)).reshape(_E, 1)
    b_scaled = b.astype(jnp.float32) * (1.0 / _H)
    t = _stage1(table.T, w_scaled, b_scaled)
    x4 = (
        x.astype(jnp.int32)
        .T.reshape(_H // 8, 8, _NW, _ROWS_PER_W)
        .transpose(0, 2, 1, 3)
    )
    y = _sc_stage2()(t, x4)
    return y.reshape(_B, 1)


# speedup vs baseline: 1.1053x; 1.0246x over previous
import functools

import jax
import jax.numpy as jnp
from jax import lax
from jax.experimental import pallas as pl
from jax.experimental.pallas import tpu as pltpu
from jax.experimental.pallas import tpu_sc as plsc

_V = 100000
_E = 64
_B = 4096
_H = 200

_NC = 2
_NS = 16
_NW = _NC * _NS
_LANES = 16

_ROWS_PER_W = _B // _NW
_GROUPS = _ROWS_PER_W // _LANES


def _stage1_body(b_ref, tblT_ref, w_ref, out_ref):
    out_ref[...] = jnp.sum(tblT_ref[...] * w_ref[...], axis=0) + b_ref[0]


def _stage1(tableT, w_scaled, b_scaled):
    blk = 25600
    return pl.pallas_call(
        _stage1_body,
        grid=(pl.cdiv(_V, blk),),
        in_specs=[
            pl.BlockSpec(memory_space=pltpu.SMEM),
            pl.BlockSpec((_E, blk), lambda i: (0, i)),
            pl.BlockSpec((_E, 1), lambda i: (0, 0)),
        ],
        out_specs=pl.BlockSpec((blk,), lambda i: (i,)),
        out_shape=jax.ShapeDtypeStruct((_V,), jnp.float32),
    )(b_scaled, tableT, w_scaled)


_CHUNK_ROWS = 64
_CHUNKS = 4


def _sc_body(t_hbm, x4_hbm, out_hbm, t_v, idx_a, idx_b, res_v, sem_t, sem_x):
    wid = lax.axis_index("s") * _NC + lax.axis_index("c")

    def chunk_src(c):
        return x4_hbm.at[:, 2 * wid + c // 2, :, pl.ds((c % 2) * _CHUNK_ROWS, _CHUNK_ROWS)]

    @pl.when(wid < _NW // 2)
    def _():
        bufs = [idx_a, idx_b]
        ct = pltpu.async_copy(t_hbm, t_v, sem_t)
        cps = [pltpu.async_copy(chunk_src(0), bufs[0], sem_x), None]

        for c in range(_CHUNKS):
            cps[c % 2].wait()
            if c == 0:
                ct.wait()
            if c + 1 < _CHUNKS:
                cps[(c + 1) % 2] = pltpu.async_copy(
                    chunk_src(c + 1), bufs[(c + 1) % 2], sem_x
                )
            buf = bufs[c % 2]

            def group_body(g, _, buf=buf):
                def tok_body(lh, acc, g=g, buf=buf):
                    for ll in range(8):
                        iv = buf[lh, ll, pl.ds(g * _LANES, _LANES)]
                        acc = acc + plsc.load_gather(t_v, [iv])
                    return acc

                z = lax.fori_loop(
                    0, _H // 8, tok_body, jnp.zeros((_LANES,), jnp.float32)
                )
                res_v[pl.ds(g * _LANES, _LANES)] = 1.0 / (1.0 + jnp.exp(-z))
                return _

            lax.fori_loop(0, _CHUNK_ROWS // _LANES, group_body, None)

            pltpu.sync_copy(
                res_v, out_hbm.at[pl.ds((2 * wid) * _ROWS_PER_W + c * _CHUNK_ROWS, _CHUNK_ROWS)]
            )


@functools.cache
def _sc_stage2():
    return pl.kernel(
        _sc_body,
        out_type=jax.ShapeDtypeStruct((_B,), jnp.float32),
        mesh=plsc.VectorSubcoreMesh(
            core_axis_name="c", subcore_axis_name="s", num_cores=_NC, num_subcores=_NS
        ),
        scratch_types=[
            pltpu.VMEM((_V,), jnp.float32),
            pltpu.VMEM((_H // 8, 8, _CHUNK_ROWS), jnp.int32),
            pltpu.VMEM((_H // 8, 8, _CHUNK_ROWS), jnp.int32),
            pltpu.VMEM((_CHUNK_ROWS,), jnp.float32),
            pltpu.SemaphoreType.DMA,
            pltpu.SemaphoreType.DMA,
        ],
        compiler_params=pltpu.CompilerParams(
            use_tc_tiling_on_sc=False, needs_layout_passes=False
        ),
    )


@jax.jit
def kernel(x, table, W, b):
    w_scaled = (W.astype(jnp.float32) * (1.0 / _H)).reshape(_E, 1)
    b_scaled = b.astype(jnp.float32) * (1.0 / _H)
    t = _stage1(table.T, w_scaled, b_scaled)
    x4 = (
        x.astype(jnp.int32)
        .T.reshape(_H // 8, 8, _NW, _ROWS_PER_W)
        .transpose(0, 2, 1, 3)
    )
    y = _sc_stage2()(t, x4)
    return y.reshape(_B, 1)
